# baseline (speedup 1.0000x reference)
import jax
import jax.numpy as jnp
from jax import lax
from jax.experimental import pallas as pl
from jax.experimental.pallas import tpu as pltpu
from jax.experimental.pallas import tpu_sc as plsc

N = 10000
D = 256
E = 160000
H = 512
NC = 2
NS = 16
NPASS = 2
CHUNK = 2560
NPAD = NC * NPASS * CHUNK
SROWS = 2688
STRIPE = SROWS // NS
LAST = CHUNK - (NS - 1) * STRIPE
EPS = E // NS
EPAD = 10240
ECH = 2000
BB = 96
NBUF = 2
NBATCH = -(-EPAD // BB)
DUMMY = CHUNK


def _sc_body(x_hbm, src_hbm, tgt_hbm, zrow_hbm,
             agg_hbm, deg_hbm,
             srcb_v, tgtb_v, fsrc_v, ftgt_v, degv_v,
             rows0_v, rows1_v, agg_s,
             gsem0, gsem1, ssem0, ssem1):
    cid = lax.axis_index("c")
    sid = lax.axis_index("s")

    rows = (rows0_v, rows1_v)
    gsems = (gsem0, gsem1)
    ssems = (ssem0, ssem1)

    dummy16 = jnp.full((16,), DUMMY, jnp.int32)
    zero16 = jnp.zeros((16,), jnp.int32)
    fzero16 = jnp.zeros((16,), jnp.float32)
    fone16 = jnp.ones((16,), jnp.float32)
    iota16 = lax.iota(jnp.int32, 16)

    for p in range(NPASS):
        base = (cid * NPASS + p) * CHUNK

        pltpu.sync_copy(zrow_hbm, agg_s.at[pl.ds(sid * STRIPE, STRIPE)])

        def zbody(i, carry):
            degv_v[pl.ds(i * 16, 16)] = fzero16
            return carry
        lax.fori_loop(0, SROWS // 16, zbody, 0)

        def fbody(i, cnt):
            t16 = tgtb_v[pl.ds(i * 16, 16)]
            s16 = srcb_v[pl.ds(i * 16, 16)]
            m = (t16 >= base) & (t16 < base + CHUNK)
            m32 = jnp.where(m, 1, 0)
            tl16 = jnp.where(m, t16 - base, DUMMY)
            pos = cnt + plsc.cumsum(m32) - 1
            plsc.store_scatter(fsrc_v, [pos], s16, mask=m)
            plsc.store_scatter(ftgt_v, [pos // BB, pos % BB], tl16, mask=m)
            plsc.addupdate_scatter(degv_v, [tl16], fone16, mask=m)
            return lax.squeeze(lax.slice(pos, (15,), (16,)), (0,)) + 1

        cnt = 0
        for c in range(EPS // ECH):
            pltpu.sync_copy(src_hbm.at[pl.ds(sid * EPS + c * ECH, ECH)],
                            srcb_v)
            pltpu.sync_copy(tgt_hbm.at[pl.ds(sid * EPS + c * ECH, ECH)],
                            tgtb_v)
            cnt = lax.fori_loop(0, ECH // 16, fbody, cnt)

        pltpu.sync_copy(degv_v.at[pl.ds(0, CHUNK)],
                        deg_hbm.at[sid, pl.ds(base, CHUNK)])

        for k in range(NBUF * BB // 16):
            pos = cnt + k * 16 + iota16
            mk = pos < EPAD
            plsc.store_scatter(fsrc_v, [pos], zero16, mask=mk)
            plsc.store_scatter(ftgt_v, [pos // BB, pos % BB], dummy16,
                               mask=mk)
        nbb = NBUF * BB
        nb = jnp.maximum(NBUF, NBUF * ((cnt + nbb - 1) // nbb))

        plsc.subcore_barrier()

        def gather(j, k):
            pltpu.async_copy(x_hbm.at[fsrc_v.at[pl.ds(j * BB, BB)]],
                             rows[k], gsems[k])

        def process(j, k):
            pltpu.make_async_copy(x_hbm.at[fsrc_v.at[pl.ds(0, BB)]],
                                  rows[k], gsems[k]).wait()
            pltpu.async_copy(rows[k], agg_s.at[ftgt_v.at[j]], ssems[k],
                             add=True)

            kn = (k + NBUF - 1) % NBUF

            @pl.when(j + NBUF - 1 < nb)
            def _():
                @pl.when(j >= 1)
                def _():
                    pltpu.make_async_copy(rows[kn], agg_s.at[ftgt_v.at[0]],
                                          ssems[kn]).wait()
                gather(j + NBUF - 1, kn)

        for k in range(NBUF - 1):
            gather(k, k)

        def bbody(j, carry):
            for k in range(NBUF):
                @pl.when(j % NBUF == k)
                def _():
                    process(j, k)
            return carry
        lax.fori_loop(0, nb, bbody, 0)

        for k in range(NBUF):
            pltpu.make_async_copy(rows[k], agg_s.at[ftgt_v.at[0]],
                                  ssems[k]).wait()

        plsc.subcore_barrier()

        @pl.when(sid < NS - 1)
        def _():
            pltpu.sync_copy(agg_s.at[pl.ds(sid * STRIPE, STRIPE)],
                            agg_hbm.at[pl.ds(base + sid * STRIPE, STRIPE)])

        @pl.when(sid == NS - 1)
        def _():
            pltpu.sync_copy(agg_s.at[pl.ds((NS - 1) * STRIPE, LAST)],
                            agg_hbm.at[pl.ds(base + (NS - 1) * STRIPE, LAST)])


@jax.jit
def _sc_aggregate(x, src, tgt, zrow):
    mesh = plsc.VectorSubcoreMesh(core_axis_name="c", subcore_axis_name="s")
    return pl.kernel(
        _sc_body,
        out_type=(jax.ShapeDtypeStruct((NPAD, D), jnp.float32),
                  jax.ShapeDtypeStruct((NS, NPAD), jnp.float32)),
        mesh=mesh,
        compiler_params=pltpu.CompilerParams(use_tc_tiling_on_sc=False,
                                             needs_layout_passes=False),
        scratch_types=[
            pltpu.VMEM((ECH,), jnp.int32),
            pltpu.VMEM((ECH,), jnp.int32),
            pltpu.VMEM((EPAD,), jnp.int32),
            pltpu.VMEM((NBATCH, BB), jnp.int32),
            pltpu.VMEM((SROWS,), jnp.float32),
            pltpu.VMEM((BB, D), jnp.float32),
            pltpu.VMEM((BB, D), jnp.float32),
            pltpu.VMEM_SHARED((SROWS, D), jnp.float32),
            pltpu.SemaphoreType.DMA,
            pltpu.SemaphoreType.DMA,
            pltpu.SemaphoreType.DMA,
            pltpu.SemaphoreType.DMA,
        ],
    )(x, src, tgt, zrow)


BN = 1024


def _mlp_body(x_ref, agg_ref, deg_ref, dl_ref, onesw_ref, w1x_ref, w1a_ref,
              w1d_ref, b1_ref, w2_ref, b2_ref, w3_ref, b3_ref, o_ref):
    degcol = lax.dot_general(deg_ref[...], onesw_ref[...],
                             (((0,), (0,)), ((), ())),
                             preferred_element_type=jnp.float32)
    deg = jnp.maximum(degcol, 1.0)
    aggn = agg_ref[...] / deg
    h = (jnp.dot(x_ref[...], w1x_ref[...], preferred_element_type=jnp.float32)
         + jnp.dot(aggn, w1a_ref[...], preferred_element_type=jnp.float32)
         + dl_ref[:, 0:1] * w1d_ref[...]
         + b1_ref[...])
    h = jnp.maximum(h, 0.0)
    h = jnp.maximum(
        jnp.dot(h, w2_ref[...], preferred_element_type=jnp.float32)
        + b2_ref[...], 0.0)
    o_ref[...] = (jnp.dot(h, w3_ref[...], preferred_element_type=jnp.float32)
                  + b3_ref[...])


@jax.jit
def _tc_mlp(x, agg, deg, dl, onesw, w1x, w1a, w1d, b1, w2, b2, w3, b3):
    grid = (-(-N // BN),)
    return pl.pallas_call(
        _mlp_body,
        grid=grid,
        in_specs=[
            pl.BlockSpec((BN, D), lambda i: (i, 0)),
            pl.BlockSpec((BN, D), lambda i: (i, 0)),
            pl.BlockSpec((NS, BN), lambda i: (0, i)),
            pl.BlockSpec((BN, 128), lambda i: (i, 0)),
            pl.BlockSpec((NS, 1), lambda i: (0, 0)),
            pl.BlockSpec((D, H), lambda i: (0, 0)),
            pl.BlockSpec((D, H), lambda i: (0, 0)),
            pl.BlockSpec((1, H), lambda i: (0, 0)),
            pl.BlockSpec((1, H), lambda i: (0, 0)),
            pl.BlockSpec((H, H), lambda i: (0, 0)),
            pl.BlockSpec((1, H), lambda i: (0, 0)),
            pl.BlockSpec((H, 8), lambda i: (0, 0)),
            pl.BlockSpec((1, 8), lambda i: (0, 0)),
        ],
        out_specs=pl.BlockSpec((BN, 8), lambda i: (i, 0)),
        out_shape=jax.ShapeDtypeStruct((N, 8), jnp.float32),
    )(x, agg, deg, dl, onesw, w1x, w1a, w1d, b1, w2, b2, w3, b3)


def kernel(x, edge_index, delta_funding, W1, b1, W2, b2, W3, b3):
    src = edge_index[0].astype(jnp.int32)
    tgt = edge_index[1].astype(jnp.int32)
    zrow = jnp.zeros((STRIPE, D), jnp.float32)
    agg, deg = _sc_aggregate(x, src, tgt, zrow)

    dl = jnp.broadcast_to(delta_funding.astype(jnp.float32)[:, None], (N, 128))
    onesw = jnp.ones((NS, 1), jnp.float32)
    w1x = W1[:, :D].T
    w1a = W1[:, D:2 * D].T
    w1d = W1[:, 2 * D:2 * D + 1].T
    b1r = b1[None, :]
    w2 = W2.T
    b2r = b2[None, :]
    w3 = jnp.zeros((H, 8), jnp.float32).at[:, :2].set(W3.T)
    b3r = jnp.zeros((1, 8), jnp.float32).at[:, :2].set(b3[None, :])
    out8 = _tc_mlp(x, agg, deg, dl, onesw, w1x, w1a, w1d, b1r, w2, b2r, w3,
                   b3r)
    return out8[:, :2]


# speedup vs baseline: 1.0359x; 1.0359x over previous
import jax
import jax.numpy as jnp
from jax import lax
from jax.experimental import pallas as pl
from jax.experimental.pallas import tpu as pltpu
from jax.experimental.pallas import tpu_sc as plsc

N = 10000
D = 256
E = 160000
H = 512
NC = 2
NS = 16
NPASS = 2
CHUNK = 2560
NPAD = NC * NPASS * CHUNK
SROWS = 2688
STRIPE = SROWS // NS
LAST = CHUNK - (NS - 1) * STRIPE
EPS = E // NS
EPAD = 10240
ECH = 2000
BB = 48
NBUF = 4
NBATCH = -(-EPAD // BB)
DUMMY = CHUNK


def _sc_body(x_hbm, src_hbm, tgt_hbm, zrow_hbm,
             agg_hbm, deg_hbm,
             srcb_v, tgtb_v, fsrc_v, ftgt_v, degv_v,
             rows0_v, rows1_v, rows2_v, rows3_v, agg_s,
             gsem0, gsem1, gsem2, gsem3, ssem0, ssem1, ssem2, ssem3):
    cid = lax.axis_index("c")
    sid = lax.axis_index("s")

    rows = (rows0_v, rows1_v, rows2_v, rows3_v)
    gsems = (gsem0, gsem1, gsem2, gsem3)
    ssems = (ssem0, ssem1, ssem2, ssem3)

    dummy16 = jnp.full((16,), DUMMY, jnp.int32)
    zero16 = jnp.zeros((16,), jnp.int32)
    fzero16 = jnp.zeros((16,), jnp.float32)
    fone16 = jnp.ones((16,), jnp.float32)
    iota16 = lax.iota(jnp.int32, 16)

    for p in range(NPASS):
        base = (cid * NPASS + p) * CHUNK

        pltpu.sync_copy(zrow_hbm, agg_s.at[pl.ds(sid * STRIPE, STRIPE)])

        def zbody(i, carry):
            degv_v[pl.ds(i * 16, 16)] = fzero16
            return carry
        lax.fori_loop(0, SROWS // 16, zbody, 0)

        def fbody(i, cnt):
            t16 = tgtb_v[pl.ds(i * 16, 16)]
            s16 = srcb_v[pl.ds(i * 16, 16)]
            m = (t16 >= base) & (t16 < base + CHUNK)
            m32 = jnp.where(m, 1, 0)
            tl16 = jnp.where(m, t16 - base, DUMMY)
            pos = cnt + plsc.cumsum(m32) - 1
            plsc.store_scatter(fsrc_v, [pos], s16, mask=m)
            plsc.store_scatter(ftgt_v, [pos // BB, pos % BB], tl16, mask=m)
            plsc.addupdate_scatter(degv_v, [tl16], fone16, mask=m)
            return lax.squeeze(lax.slice(pos, (15,), (16,)), (0,)) + 1

        cnt = 0
        for c in range(EPS // ECH):
            pltpu.sync_copy(src_hbm.at[pl.ds(sid * EPS + c * ECH, ECH)],
                            srcb_v)
            pltpu.sync_copy(tgt_hbm.at[pl.ds(sid * EPS + c * ECH, ECH)],
                            tgtb_v)
            cnt = lax.fori_loop(0, ECH // 16, fbody, cnt)

        pltpu.sync_copy(degv_v.at[pl.ds(0, CHUNK)],
                        deg_hbm.at[sid, pl.ds(base, CHUNK)])

        for k in range(NBUF * BB // 16):
            pos = cnt + k * 16 + iota16
            mk = pos < EPAD
            plsc.store_scatter(fsrc_v, [pos], zero16, mask=mk)
            plsc.store_scatter(ftgt_v, [pos // BB, pos % BB], dummy16,
                               mask=mk)
        nbb = NBUF * BB
        nb = jnp.maximum(NBUF, NBUF * ((cnt + nbb - 1) // nbb))

        plsc.subcore_barrier()

        def gather(j, k):
            pltpu.async_copy(x_hbm.at[fsrc_v.at[pl.ds(j * BB, BB)]],
                             rows[k], gsems[k])

        def process(j, k):
            pltpu.make_async_copy(x_hbm.at[fsrc_v.at[pl.ds(0, BB)]],
                                  rows[k], gsems[k]).wait()
            pltpu.async_copy(rows[k], agg_s.at[ftgt_v.at[j]], ssems[k],
                             add=True)

            kn = (k + NBUF - 1) % NBUF

            @pl.when(j + NBUF - 1 < nb)
            def _():
                @pl.when(j >= 1)
                def _():
                    pltpu.make_async_copy(rows[kn], agg_s.at[ftgt_v.at[0]],
                                          ssems[kn]).wait()
                gather(j + NBUF - 1, kn)

        for k in range(NBUF - 1):
            gather(k, k)

        def bbody(j, carry):
            for k in range(NBUF):
                @pl.when(j % NBUF == k)
                def _():
                    process(j, k)
            return carry
        lax.fori_loop(0, nb, bbody, 0)

        for k in range(NBUF):
            pltpu.make_async_copy(rows[k], agg_s.at[ftgt_v.at[0]],
                                  ssems[k]).wait()

        plsc.subcore_barrier()

        @pl.when(sid < NS - 1)
        def _():
            pltpu.sync_copy(agg_s.at[pl.ds(sid * STRIPE, STRIPE)],
                            agg_hbm.at[pl.ds(base + sid * STRIPE, STRIPE)])

        @pl.when(sid == NS - 1)
        def _():
            pltpu.sync_copy(agg_s.at[pl.ds((NS - 1) * STRIPE, LAST)],
                            agg_hbm.at[pl.ds(base + (NS - 1) * STRIPE, LAST)])


@jax.jit
def _sc_aggregate(x, src, tgt, zrow):
    mesh = plsc.VectorSubcoreMesh(core_axis_name="c", subcore_axis_name="s")
    return pl.kernel(
        _sc_body,
        out_type=(jax.ShapeDtypeStruct((NPAD, D), jnp.float32),
                  jax.ShapeDtypeStruct((NS, NPAD), jnp.float32)),
        mesh=mesh,
        compiler_params=pltpu.CompilerParams(use_tc_tiling_on_sc=False,
                                             needs_layout_passes=False),
        scratch_types=[
            pltpu.VMEM((ECH,), jnp.int32),
            pltpu.VMEM((ECH,), jnp.int32),
            pltpu.VMEM((EPAD,), jnp.int32),
            pltpu.VMEM((NBATCH, BB), jnp.int32),
            pltpu.VMEM((SROWS,), jnp.float32),
            pltpu.VMEM((BB, D), jnp.float32),
            pltpu.VMEM((BB, D), jnp.float32),
            pltpu.VMEM((BB, D), jnp.float32),
            pltpu.VMEM((BB, D), jnp.float32),
            pltpu.VMEM_SHARED((SROWS, D), jnp.float32),
            pltpu.SemaphoreType.DMA,
            pltpu.SemaphoreType.DMA,
            pltpu.SemaphoreType.DMA,
            pltpu.SemaphoreType.DMA,
            pltpu.SemaphoreType.DMA,
            pltpu.SemaphoreType.DMA,
            pltpu.SemaphoreType.DMA,
            pltpu.SemaphoreType.DMA,
        ],
    )(x, src, tgt, zrow)


BN = 1024


def _mlp_body(x_ref, agg_ref, deg_ref, dl_ref, onesw_ref, w1x_ref, w1a_ref,
              w1d_ref, b1_ref, w2_ref, b2_ref, w3_ref, b3_ref, o_ref):
    degcol = lax.dot_general(deg_ref[...], onesw_ref[...],
                             (((0,), (0,)), ((), ())),
                             preferred_element_type=jnp.float32)
    deg = jnp.maximum(degcol, 1.0)
    aggn = agg_ref[...] / deg
    h = (jnp.dot(x_ref[...], w1x_ref[...], preferred_element_type=jnp.float32)
         + jnp.dot(aggn, w1a_ref[...], preferred_element_type=jnp.float32)
         + dl_ref[:, 0:1] * w1d_ref[...]
         + b1_ref[...])
    h = jnp.maximum(h, 0.0)
    h = jnp.maximum(
        jnp.dot(h, w2_ref[...], preferred_element_type=jnp.float32)
        + b2_ref[...], 0.0)
    o_ref[...] = (jnp.dot(h, w3_ref[...], preferred_element_type=jnp.float32)
                  + b3_ref[...])


@jax.jit
def _tc_mlp(x, agg, deg, dl, onesw, w1x, w1a, w1d, b1, w2, b2, w3, b3):
    grid = (-(-N // BN),)
    return pl.pallas_call(
        _mlp_body,
        grid=grid,
        in_specs=[
            pl.BlockSpec((BN, D), lambda i: (i, 0)),
            pl.BlockSpec((BN, D), lambda i: (i, 0)),
            pl.BlockSpec((NS, BN), lambda i: (0, i)),
            pl.BlockSpec((BN, 128), lambda i: (i, 0)),
            pl.BlockSpec((NS, 1), lambda i: (0, 0)),
            pl.BlockSpec((D, H), lambda i: (0, 0)),
            pl.BlockSpec((D, H), lambda i: (0, 0)),
            pl.BlockSpec((1, H), lambda i: (0, 0)),
            pl.BlockSpec((1, H), lambda i: (0, 0)),
            pl.BlockSpec((H, H), lambda i: (0, 0)),
            pl.BlockSpec((1, H), lambda i: (0, 0)),
            pl.BlockSpec((H, 8), lambda i: (0, 0)),
            pl.BlockSpec((1, 8), lambda i: (0, 0)),
        ],
        out_specs=pl.BlockSpec((BN, 8), lambda i: (i, 0)),
        out_shape=jax.ShapeDtypeStruct((N, 8), jnp.float32),
    )(x, agg, deg, dl, onesw, w1x, w1a, w1d, b1, w2, b2, w3, b3)


def kernel(x, edge_index, delta_funding, W1, b1, W2, b2, W3, b3):
    src = edge_index[0].astype(jnp.int32)
    tgt = edge_index[1].astype(jnp.int32)
    zrow = jnp.zeros((STRIPE, D), jnp.float32)
    agg, deg = _sc_aggregate(x, src, tgt, zrow)

    dl = jnp.broadcast_to(delta_funding.astype(jnp.float32)[:, None], (N, 128))
    onesw = jnp.ones((NS, 1), jnp.float32)
    w1x = W1[:, :D].T
    w1a = W1[:, D:2 * D].T
    w1d = W1[:, 2 * D:2 * D + 1].T
    b1r = b1[None, :]
    w2 = W2.T
    b2r = b2[None, :]
    w3 = jnp.zeros((H, 8), jnp.float32).at[:, :2].set(W3.T)
    b3r = jnp.zeros((1, 8), jnp.float32).at[:, :2].set(b3[None, :])
    out8 = _tc_mlp(x, agg, deg, dl, onesw, w1x, w1a, w1d, b1r, w2, b2r, w3,
                   b3r)
    return out8[:, :2]


# speedup vs baseline: 1.0379x; 1.0020x over previous
import jax
import jax.numpy as jnp
from jax import lax
from jax.experimental import pallas as pl
from jax.experimental.pallas import tpu as pltpu
from jax.experimental.pallas import tpu_sc as plsc

N = 10000
D = 256
E = 160000
H = 512
NC = 2
NS = 16
NPASS = 2
CHUNK = 2560
NPAD = NC * NPASS * CHUNK
SROWS = 2688
STRIPE = SROWS // NS
LAST = CHUNK - (NS - 1) * STRIPE
EPS = E // NS
EPAD = 10240
ECH = 2000
BB = 48
NBUF = 4
NBATCH = -(-EPAD // BB)
DUMMY = CHUNK


def _sc_body(x_hbm, src_hbm, tgt_hbm, zrow_hbm,
             agg_hbm, deg_hbm,
             srcb_v, tgtb_v, fsrc_v, ftgt_v, degv_v,
             rows0_v, rows1_v, rows2_v, rows3_v, agg_s,
             gsem0, gsem1, gsem2, gsem3, ssem0, ssem1, ssem2, ssem3):
    cid = lax.axis_index("c")
    sid = lax.axis_index("s")

    rows = (rows0_v, rows1_v, rows2_v, rows3_v)
    gsems = (gsem0, gsem1, gsem2, gsem3)
    ssems = (ssem0, ssem1, ssem2, ssem3)

    zero16 = jnp.zeros((16,), jnp.int32)
    dummy16 = zero16 + CHUNK + sid * 8
    fzero16 = jnp.zeros((16,), jnp.float32)
    fone16 = jnp.ones((16,), jnp.float32)
    iota16 = lax.iota(jnp.int32, 16)

    for p in range(NPASS):
        base = (cid * NPASS + p) * CHUNK

        pltpu.sync_copy(zrow_hbm, agg_s.at[pl.ds(sid * STRIPE, STRIPE)])

        def zbody(i, carry):
            degv_v[pl.ds(i * 16, 16)] = fzero16
            return carry
        lax.fori_loop(0, SROWS // 16, zbody, 0)

        def fbody(i, cnt):
            t16 = tgtb_v[pl.ds(i * 16, 16)]
            s16 = srcb_v[pl.ds(i * 16, 16)]
            m = (t16 >= base) & (t16 < base + CHUNK)
            m32 = jnp.where(m, 1, 0)
            tl16 = jnp.where(m, t16 - base, dummy16)
            pos = cnt + plsc.cumsum(m32) - 1
            plsc.store_scatter(fsrc_v, [pos], s16, mask=m)
            plsc.store_scatter(ftgt_v, [pos // BB, pos % BB], tl16, mask=m)
            plsc.addupdate_scatter(degv_v, [tl16], fone16, mask=m)
            return lax.squeeze(lax.slice(pos, (15,), (16,)), (0,)) + 1

        cnt = 0
        for c in range(EPS // ECH):
            pltpu.sync_copy(src_hbm.at[pl.ds(sid * EPS + c * ECH, ECH)],
                            srcb_v)
            pltpu.sync_copy(tgt_hbm.at[pl.ds(sid * EPS + c * ECH, ECH)],
                            tgtb_v)
            cnt = lax.fori_loop(0, ECH // 16, fbody, cnt)

        pltpu.sync_copy(degv_v.at[pl.ds(0, CHUNK)],
                        deg_hbm.at[sid, pl.ds(base, CHUNK)])

        for k in range(NBUF * BB // 16):
            pos = cnt + k * 16 + iota16
            mk = pos < EPAD
            plsc.store_scatter(fsrc_v, [pos], zero16, mask=mk)
            plsc.store_scatter(ftgt_v, [pos // BB, pos % BB], dummy16,
                               mask=mk)
        nbb = NBUF * BB
        nb = jnp.maximum(NBUF, NBUF * ((cnt + nbb - 1) // nbb))

        plsc.subcore_barrier()

        def gather(j, k):
            pltpu.async_copy(x_hbm.at[fsrc_v.at[pl.ds(j * BB, BB)]],
                             rows[k], gsems[k])

        def process(j, k):
            pltpu.make_async_copy(x_hbm.at[fsrc_v.at[pl.ds(0, BB)]],
                                  rows[k], gsems[k]).wait()
            pltpu.async_copy(rows[k], agg_s.at[ftgt_v.at[j]], ssems[k],
                             add=True)

            kn = (k + NBUF - 1) % NBUF

            @pl.when(j + NBUF - 1 < nb)
            def _():
                @pl.when(j >= 1)
                def _():
                    pltpu.make_async_copy(rows[kn], agg_s.at[ftgt_v.at[0]],
                                          ssems[kn]).wait()
                gather(j + NBUF - 1, kn)

        for k in range(NBUF - 1):
            gather(k, k)

        def bbody(j, carry):
            for k in range(NBUF):
                @pl.when(j % NBUF == k)
                def _():
                    process(j, k)
            return carry
        lax.fori_loop(0, nb, bbody, 0)

        for k in range(NBUF):
            pltpu.make_async_copy(rows[k], agg_s.at[ftgt_v.at[0]],
                                  ssems[k]).wait()

        plsc.subcore_barrier()

        @pl.when(sid < NS - 1)
        def _():
            pltpu.sync_copy(agg_s.at[pl.ds(sid * STRIPE, STRIPE)],
                            agg_hbm.at[pl.ds(base + sid * STRIPE, STRIPE)])

        @pl.when(sid == NS - 1)
        def _():
            pltpu.sync_copy(agg_s.at[pl.ds((NS - 1) * STRIPE, LAST)],
                            agg_hbm.at[pl.ds(base + (NS - 1) * STRIPE, LAST)])


@jax.jit
def _sc_aggregate(x, src, tgt, zrow):
    mesh = plsc.VectorSubcoreMesh(core_axis_name="c", subcore_axis_name="s")
    return pl.kernel(
        _sc_body,
        out_type=(jax.ShapeDtypeStruct((NPAD, D), jnp.float32),
                  jax.ShapeDtypeStruct((NS, NPAD), jnp.float32)),
        mesh=mesh,
        compiler_params=pltpu.CompilerParams(use_tc_tiling_on_sc=False,
                                             needs_layout_passes=False),
        scratch_types=[
            pltpu.VMEM((ECH,), jnp.int32),
            pltpu.VMEM((ECH,), jnp.int32),
            pltpu.VMEM((EPAD,), jnp.int32),
            pltpu.VMEM((NBATCH, BB), jnp.int32),
            pltpu.VMEM((SROWS,), jnp.float32),
            pltpu.VMEM((BB, D), jnp.float32),
            pltpu.VMEM((BB, D), jnp.float32),
            pltpu.VMEM((BB, D), jnp.float32),
            pltpu.VMEM((BB, D), jnp.float32),
            pltpu.VMEM_SHARED((SROWS, D), jnp.float32),
            pltpu.SemaphoreType.DMA,
            pltpu.SemaphoreType.DMA,
            pltpu.SemaphoreType.DMA,
            pltpu.SemaphoreType.DMA,
            pltpu.SemaphoreType.DMA,
            pltpu.SemaphoreType.DMA,
            pltpu.SemaphoreType.DMA,
            pltpu.SemaphoreType.DMA,
        ],
    )(x, src, tgt, zrow)


BN = 1024


def _mlp_body(x_ref, agg_ref, deg_ref, dl_ref, onesw_ref, w1x_ref, w1a_ref,
              w1d_ref, b1_ref, w2_ref, b2_ref, w3_ref, b3_ref, o_ref):
    degcol = lax.dot_general(deg_ref[...], onesw_ref[...],
                             (((0,), (0,)), ((), ())),
                             preferred_element_type=jnp.float32)
    deg = jnp.maximum(degcol, 1.0)
    aggn = agg_ref[...] / deg
    h = (jnp.dot(x_ref[...], w1x_ref[...], preferred_element_type=jnp.float32)
         + jnp.dot(aggn, w1a_ref[...], preferred_element_type=jnp.float32)
         + dl_ref[:, 0:1] * w1d_ref[...]
         + b1_ref[...])
    h = jnp.maximum(h, 0.0)
    h = jnp.maximum(
        jnp.dot(h, w2_ref[...], preferred_element_type=jnp.float32)
        + b2_ref[...], 0.0)
    o_ref[...] = (jnp.dot(h, w3_ref[...], preferred_element_type=jnp.float32)
                  + b3_ref[...])


@jax.jit
def _tc_mlp(x, agg, deg, dl, onesw, w1x, w1a, w1d, b1, w2, b2, w3, b3):
    grid = (-(-N // BN),)
    return pl.pallas_call(
        _mlp_body,
        grid=grid,
        in_specs=[
            pl.BlockSpec((BN, D), lambda i: (i, 0)),
            pl.BlockSpec((BN, D), lambda i: (i, 0)),
            pl.BlockSpec((NS, BN), lambda i: (0, i)),
            pl.BlockSpec((BN, 128), lambda i: (i, 0)),
            pl.BlockSpec((NS, 1), lambda i: (0, 0)),
            pl.BlockSpec((D, H), lambda i: (0, 0)),
            pl.BlockSpec((D, H), lambda i: (0, 0)),
            pl.BlockSpec((1, H), lambda i: (0, 0)),
            pl.BlockSpec((1, H), lambda i: (0, 0)),
            pl.BlockSpec((H, H), lambda i: (0, 0)),
            pl.BlockSpec((1, H), lambda i: (0, 0)),
            pl.BlockSpec((H, 8), lambda i: (0, 0)),
            pl.BlockSpec((1, 8), lambda i: (0, 0)),
        ],
        out_specs=pl.BlockSpec((BN, 8), lambda i: (i, 0)),
        out_shape=jax.ShapeDtypeStruct((N, 8), jnp.float32),
    )(x, agg, deg, dl, onesw, w1x, w1a, w1d, b1, w2, b2, w3, b3)


def kernel(x, edge_index, delta_funding, W1, b1, W2, b2, W3, b3):
    src = edge_index[0].astype(jnp.int32)
    tgt = edge_index[1].astype(jnp.int32)
    zrow = jnp.zeros((STRIPE, D), jnp.float32)
    agg, deg = _sc_aggregate(x, src, tgt, zrow)

    dl = jnp.broadcast_to(delta_funding.astype(jnp.float32)[:, None], (N, 128))
    onesw = jnp.ones((NS, 1), jnp.float32)
    w1x = W1[:, :D].T
    w1a = W1[:, D:2 * D].T
    w1d = W1[:, 2 * D:2 * D + 1].T
    b1r = b1[None, :]
    w2 = W2.T
    b2r = b2[None, :]
    w3 = jnp.zeros((H, 8), jnp.float32).at[:, :2].set(W3.T)
    b3r = jnp.zeros((1, 8), jnp.float32).at[:, :2].set(b3[None, :])
    out8 = _tc_mlp(x, agg, deg, dl, onesw, w1x, w1a, w1d, b1r, w2, b2r, w3,
                   b3r)
    return out8[:, :2]
